# baseline (speedup 1.0000x reference)
import jax
import jax.numpy as jnp
from jax import lax
from jax.experimental import pallas as pl
from jax.experimental.pallas import tpu as pltpu
from jax.experimental.pallas import tpu_sc as plsc

VOCAB_N = 1000000
EMBED_N = 64
BATCH_N = 4096
X_MAX_C = 100.0
ALPHA_C = 0.75

_NUM_WORKERS = 32
_LANES = 16
_NSLOT = 4

_CHUNKS = (1536, 1536, 1024)
_ROW_BLOCK = 512


def _make_sc_body(bpw, off, with_bias):
    niter = bpw // _NSLOT

    def body(f_idx, c_idx, fe_t, ce_t, fb_t, cb_t, *rest):
        if with_bias:
            (p_out, b_out,
             fi_v, ci_v,
             feb0, feb1, feb2, feb3, ceb0, ceb1, ceb2, ceb3,
             p_v,
             bfi, bci, fbB, cbB, b_v,
             sem0, sem1, sem2, sem3, semb) = rest
        else:
            (p_out,
             fi_v, ci_v,
             feb0, feb1, feb2, feb3, ceb0, ceb1, ceb2, ceb3,
             p_v,
             sem0, sem1, sem2, sem3) = rest
        febs = (feb0, feb1, feb2, feb3)
        cebs = (ceb0, ceb1, ceb2, ceb3)
        sems = (sem0, sem1, sem2, sem3)

        wid = lax.axis_index("s") * 2 + lax.axis_index("c")
        base = off + wid * bpw
        pltpu.sync_copy(f_idx.at[pl.ds(base, bpw)], fi_v.at[pl.ds(0, bpw)])
        pltpu.sync_copy(c_idx.at[pl.ds(base, bpw)], ci_v.at[pl.ds(0, bpw)])

        lanes = lax.iota(jnp.int32, _LANES)

        if with_bias:
            bbase = wid * 128
            pltpu.sync_copy(f_idx.at[pl.ds(bbase, 128)], bfi)
            pltpu.sync_copy(c_idx.at[pl.ds(bbase, 128)], bci)
            wins_f = [bfi[pl.ds(g * _LANES, _LANES)] for g in range(8)]
            wins_c = [bci[pl.ds(g * _LANES, _LANES)] for g in range(8)]
            for i in range(128):
                jf = wins_f[i // 16][i % 16]
                jc = wins_c[i // 16][i % 16]
                jtf = pl.multiple_of((jf // 128) * 128, 128)
                jtc = pl.multiple_of((jc // 128) * 128, 128)
                pltpu.async_copy(fb_t.at[0, pl.ds(jtf, 128)], fbB.at[i], semb)
                pltpu.async_copy(cb_t.at[0, pl.ds(jtc, 128)], cbB.at[i], semb)

        def fire(s, jf, jc):
            jtf = pl.multiple_of((jf // 128) * 128, 128)
            jtc = pl.multiple_of((jc // 128) * 128, 128)
            pltpu.async_copy(fe_t.at[:, pl.ds(jtf, 128)], febs[s], sems[s])
            pltpu.async_copy(ce_t.at[:, pl.ds(jtc, 128)], cebs[s], sems[s])

        def wait_slot(s):
            pltpu.make_async_copy(
                fe_t.at[:, pl.ds(0, 128)], febs[s], sems[s]).wait()
            pltpu.make_async_copy(
                ce_t.at[:, pl.ds(0, 128)], cebs[s], sems[s]).wait()

        win_f0 = fi_v[pl.ds(0, _LANES)]
        win_c0 = ci_v[pl.ds(0, _LANES)]
        for s in range(_NSLOT):
            fire(s, win_f0[s], win_c0[s])

        def step(t, carry):
            pvec = carry
            win_f = fi_v[pl.ds(t * _NSLOT, _LANES)]
            win_c = ci_v[pl.ds(t * _NSLOT, _LANES)]
            win_fn = fi_v[pl.ds(t * _NSLOT + _NSLOT, _LANES)]
            win_cn = ci_v[pl.ds(t * _NSLOT + _NSLOT, _LANES)]
            lbase = (t % 4) * _NSLOT
            for s in range(_NSLOT):
                wait_slot(s)
                jl_f = win_f[s] % 128
                jl_c = win_c[s] % 128
                colf = jnp.zeros((_LANES,), jnp.int32) + jl_f
                colc = jnp.zeros((_LANES,), jnp.int32) + jl_c
                acc = jnp.zeros((_LANES,), jnp.float32)
                for g in range(EMBED_N // _LANES):
                    rows = lanes + (g * _LANES)
                    fv = plsc.load_gather(febs[s], [rows, colf])
                    cv = plsc.load_gather(cebs[s], [rows, colc])
                    acc = acc + fv * cv
                p_s = jnp.sum(acc)

                @pl.when(t < niter - 1)
                def _():
                    fire(s, win_fn[s], win_cn[s])

                pvec = jnp.where(lanes == (lbase + s), p_s, pvec)

            @pl.when(t % 4 == 3)
            def _():
                p_v[pl.ds((t // 4) * _LANES, _LANES)] = pvec

            done = (jnp.zeros((_LANES,), jnp.int32) + (t % 4)) == 3
            return jnp.where(done, 0.0, pvec)

        lax.fori_loop(0, niter, step, jnp.zeros((_LANES,), jnp.float32))
        pltpu.sync_copy(p_v.at[pl.ds(0, bpw)], p_out.at[pl.ds(base - off, bpw)])

        if with_bias:
            for i in range(128):
                pltpu.make_async_copy(
                    fb_t.at[0, pl.ds(0, 128)], fbB.at[i], semb).wait()
                pltpu.make_async_copy(
                    cb_t.at[0, pl.ds(0, 128)], cbB.at[i], semb).wait()
            for g in range(8):
                jlf = wins_f[g] % 128
                jlc = wins_c[g] % 128
                rows = lanes + g * _LANES
                fbv = plsc.load_gather(fbB, [rows, jlf])
                cbv = plsc.load_gather(cbB, [rows, jlc])
                b_v[pl.ds(g * _LANES, _LANES)] = fbv + cbv
            pltpu.sync_copy(b_v, b_out.at[pl.ds(bbase, 128)])

    return body


def _make_sc_call(chunk, off, with_bias):
    bpw = chunk // _NUM_WORKERS
    emb_buf = pltpu.VMEM((EMBED_N, 128), jnp.float32)
    idx_buf = pltpu.VMEM((bpw + _LANES,), jnp.int32)
    if with_bias:
        out_type = (jax.ShapeDtypeStruct((chunk,), jnp.float32),
                    jax.ShapeDtypeStruct((BATCH_N,), jnp.float32))
        extra = ([pltpu.VMEM((128,), jnp.int32)] * 2
                 + [pltpu.VMEM((128, 128), jnp.float32)] * 2
                 + [pltpu.VMEM((128,), jnp.float32)]
                 + [pltpu.SemaphoreType.DMA] * 5)
    else:
        out_type = jax.ShapeDtypeStruct((chunk,), jnp.float32)
        extra = [pltpu.SemaphoreType.DMA] * 4
    return pl.kernel(
        _make_sc_body(bpw, off, with_bias),
        out_type=out_type,
        mesh=plsc.VectorSubcoreMesh(core_axis_name="c", subcore_axis_name="s"),
        compiler_params=pltpu.CompilerParams(
            needs_layout_passes=False, skip_device_barrier=True),
        scratch_types=(
            [idx_buf] * 2 + [emb_buf] * 8
            + [pltpu.VMEM((bpw,), jnp.float32)]
            + extra
        ),
    )


def _loss_stripe(prev_ref, p_ref, c_ref, b_ref, o_ref):
    c = c_ref[...]
    a = p_ref[...] - jnp.log(c)
    w = jnp.minimum(jnp.exp(ALPHA_C * jnp.log(c * (1.0 / X_MAX_C))), 1.0)
    s = a + b_ref[...]
    o_ref[...] = w * (s * s)


def kernel(focal_input, context_input, cooccurance_count,
           focal_embedding, context_embedding, focal_biases, context_biases):
    fi = focal_input.astype(jnp.int32)
    ci = context_input.astype(jnp.int32)
    cooc = cooccurance_count.astype(jnp.float32)
    fe_t = focal_embedding.T
    ce_t = context_embedding.T
    fb_t = focal_biases.T
    cb_t = context_biases.T

    ps = []
    b = None
    off = 0
    for c_i, chunk in enumerate(_CHUNKS):
        call = _make_sc_call(chunk, off, with_bias=(c_i == 0))
        if c_i == 0:
            p_c, b = call(fi, ci, fe_t, ce_t, fb_t, cb_t)
        else:
            p_c = call(fi, ci, fe_t, ce_t, fb_t, cb_t)
        ps.append(p_c)
        off += chunk

    b2d = b.reshape(BATCH_N, 1)
    out = None
    off = 0
    for c_i, chunk in enumerate(_CHUNKS):
        cw = chunk
        col_block = off // cw
        grid = (BATCH_N // _ROW_BLOCK,)
        in_specs = [
            pl.BlockSpec((1, cw), lambda i: (0, 0)),
            pl.BlockSpec((1, cw), lambda i: (0, 0)),
            pl.BlockSpec((_ROW_BLOCK, 1), lambda i: (i, 0)),
        ]
        out_spec = pl.BlockSpec(
            (_ROW_BLOCK, cw), lambda i, cb=col_block: (i, cb))
        p2d = ps[c_i].reshape(1, cw)
        c2d = lax.slice(cooc, (off,), (off + cw,)).reshape(1, cw)
        if c_i == 0:
            out = pl.pallas_call(
                lambda p_ref, c_ref, b_ref, o_ref: _loss_stripe(
                    None, p_ref, c_ref, b_ref, o_ref),
                grid=grid,
                in_specs=in_specs,
                out_specs=out_spec,
                out_shape=jax.ShapeDtypeStruct((BATCH_N, BATCH_N), jnp.float32),
            )(p2d, c2d, b2d)
        else:
            out = pl.pallas_call(
                _loss_stripe,
                grid=grid,
                in_specs=[pl.BlockSpec(memory_space=pltpu.MemorySpace.HBM)]
                + in_specs,
                out_specs=out_spec,
                out_shape=jax.ShapeDtypeStruct((BATCH_N, BATCH_N), jnp.float32),
                input_output_aliases={0: 0},
            )(out, p2d, c2d, b2d)
        off += chunk
    return out


# speedup vs baseline: 1.1292x; 1.1292x over previous
import jax
import jax.numpy as jnp
from jax import lax
from jax.experimental import pallas as pl
from jax.experimental.pallas import tpu as pltpu
from jax.experimental.pallas import tpu_sc as plsc

VOCAB_N = 1000000
EMBED_N = 64
BATCH_N = 4096
X_MAX_C = 100.0
ALPHA_C = 0.75

_NUM_WORKERS = 32
_BPW = BATCH_N // _NUM_WORKERS
_LANES = 16
_NSLOT = 4
_NITER = _BPW // _NSLOT

_ROW_BLOCK = 512


def _sc_body(f_idx, c_idx, fe_t, ce_t, fb_t, cb_t,
             p_out, b_out,
             fi_v, ci_v,
             feb0, feb1, feb2, feb3, ceb0, ceb1, ceb2, ceb3,
             fbb0, fbb1, fbb2, fbb3, cbb0, cbb1, cbb2, cbb3,
             p_v, b_v, sem0, sem1, sem2, sem3):
    febs = (feb0, feb1, feb2, feb3)
    cebs = (ceb0, ceb1, ceb2, ceb3)
    fbbs = (fbb0, fbb1, fbb2, fbb3)
    cbbs = (cbb0, cbb1, cbb2, cbb3)
    sems = (sem0, sem1, sem2, sem3)

    wid = lax.axis_index("s") * 2 + lax.axis_index("c")
    base = wid * _BPW
    pltpu.sync_copy(f_idx.at[pl.ds(base, _BPW)], fi_v.at[pl.ds(0, _BPW)])
    pltpu.sync_copy(c_idx.at[pl.ds(base, _BPW)], ci_v.at[pl.ds(0, _BPW)])

    lanes = lax.iota(jnp.int32, _LANES)

    def fire(s, jf, jc):
        jtf = pl.multiple_of((jf // 128) * 128, 128)
        jtc = pl.multiple_of((jc // 128) * 128, 128)
        pltpu.async_copy(fe_t.at[:, pl.ds(jtf, 128)], febs[s], sems[s])
        pltpu.async_copy(ce_t.at[:, pl.ds(jtc, 128)], cebs[s], sems[s])
        pltpu.async_copy(fb_t.at[0, pl.ds(jtf, 128)], fbbs[s], sems[s])
        pltpu.async_copy(cb_t.at[0, pl.ds(jtc, 128)], cbbs[s], sems[s])

    def wait_slot(s):
        pltpu.make_async_copy(fe_t.at[:, pl.ds(0, 128)], febs[s], sems[s]).wait()
        pltpu.make_async_copy(ce_t.at[:, pl.ds(0, 128)], cebs[s], sems[s]).wait()
        pltpu.make_async_copy(fb_t.at[0, pl.ds(0, 128)], fbbs[s], sems[s]).wait()
        pltpu.make_async_copy(cb_t.at[0, pl.ds(0, 128)], cbbs[s], sems[s]).wait()

    win_f0 = fi_v[pl.ds(0, _LANES)]
    win_c0 = ci_v[pl.ds(0, _LANES)]
    for s in range(_NSLOT):
        fire(s, win_f0[s], win_c0[s])

    def step(t, carry):
        pvec, bvec = carry
        win_f = fi_v[pl.ds(t * _NSLOT, _LANES)]
        win_c = ci_v[pl.ds(t * _NSLOT, _LANES)]
        win_fn = fi_v[pl.ds(t * _NSLOT + _NSLOT, _LANES)]
        win_cn = ci_v[pl.ds(t * _NSLOT + _NSLOT, _LANES)]
        lbase = (t % 4) * _NSLOT
        for s in range(_NSLOT):
            wait_slot(s)
            jl_f = win_f[s] % 128
            jl_c = win_c[s] % 128
            colf = jnp.zeros((_LANES,), jnp.int32) + jl_f
            colc = jnp.zeros((_LANES,), jnp.int32) + jl_c
            acc = jnp.zeros((_LANES,), jnp.float32)
            for g in range(EMBED_N // _LANES):
                rows = lanes + (g * _LANES)
                fv = plsc.load_gather(febs[s], [rows, colf])
                cv = plsc.load_gather(cebs[s], [rows, colc])
                acc = acc + fv * cv
            p_s = jnp.sum(acc)
            fbv = plsc.load_gather(fbbs[s], [colf])
            cbv = plsc.load_gather(cbbs[s], [colc])
            b_s = fbv[0] + cbv[0]

            @pl.when(t < _NITER - 1)
            def _():
                fire(s, win_fn[s], win_cn[s])

            msk = lanes == (lbase + s)
            pvec = jnp.where(msk, p_s, pvec)
            bvec = jnp.where(msk, b_s, bvec)

        @pl.when(t % 4 == 3)
        def _():
            p_v[pl.ds((t // 4) * _LANES, _LANES)] = pvec
            b_v[pl.ds((t // 4) * _LANES, _LANES)] = bvec

        done = (jnp.zeros((_LANES,), jnp.int32) + (t % 4)) == 3
        pvec = jnp.where(done, 0.0, pvec)
        bvec = jnp.where(done, 0.0, bvec)
        return pvec, bvec

    lax.fori_loop(0, _NITER, step,
                  (jnp.zeros((_LANES,), jnp.float32),
                   jnp.zeros((_LANES,), jnp.float32)))

    pltpu.sync_copy(p_v, p_out.at[pl.ds(base, _BPW)])
    pltpu.sync_copy(b_v, b_out.at[pl.ds(base, _BPW)])


def _loss_body(p_ref, c_ref, b_ref, o_ref):
    c = c_ref[...]
    a = p_ref[...] - jnp.log(c)
    w = jnp.minimum(jnp.exp(ALPHA_C * jnp.log(c * (1.0 / X_MAX_C))), 1.0)
    s = a + b_ref[...]
    o_ref[...] = w * (s * s)


def kernel(focal_input, context_input, cooccurance_count,
           focal_embedding, context_embedding, focal_biases, context_biases):
    fi = focal_input.astype(jnp.int32)
    ci = context_input.astype(jnp.int32)
    cooc = cooccurance_count.astype(jnp.float32)

    emb_buf = pltpu.VMEM((EMBED_N, 128), jnp.float32)
    bias_buf = pltpu.VMEM((128,), jnp.float32)
    sc_gather = pl.kernel(
        _sc_body,
        out_type=(
            jax.ShapeDtypeStruct((BATCH_N,), jnp.float32),
            jax.ShapeDtypeStruct((BATCH_N,), jnp.float32),
        ),
        mesh=plsc.VectorSubcoreMesh(core_axis_name="c", subcore_axis_name="s"),
        compiler_params=pltpu.CompilerParams(needs_layout_passes=False),
        scratch_types=(
            [pltpu.VMEM((_BPW + _LANES,), jnp.int32)] * 2
            + [emb_buf] * 8
            + [bias_buf] * 8
            + [pltpu.VMEM((_BPW,), jnp.float32)] * 2
            + [pltpu.SemaphoreType.DMA] * 4
        ),
    )
    p, b = sc_gather(fi, ci, focal_embedding.T, context_embedding.T,
                     focal_biases.T, context_biases.T)

    out = pl.pallas_call(
        _loss_body,
        grid=(BATCH_N // _ROW_BLOCK,),
        in_specs=[
            pl.BlockSpec((1, BATCH_N), lambda i: (0, 0)),
            pl.BlockSpec((1, BATCH_N), lambda i: (0, 0)),
            pl.BlockSpec((_ROW_BLOCK, 1), lambda i: (i, 0)),
        ],
        out_specs=pl.BlockSpec((_ROW_BLOCK, BATCH_N), lambda i: (i, 0)),
        out_shape=jax.ShapeDtypeStruct((BATCH_N, BATCH_N), jnp.float32),
    )(p.reshape(1, BATCH_N), cooc.reshape(1, BATCH_N), b.reshape(BATCH_N, 1))
    return out
